# Initial kernel scaffold; baseline (speedup 1.0000x reference)
import jax
import jax.numpy as jnp
from jax.experimental import pallas as pl


def kernel(fix_image_feature, moving_image_feature, fixed_points, positive_points, negative_points):
    raise NotImplementedError("write your pallas kernel here")


# speedup vs baseline: 1.0132x; 1.0132x over previous
import functools

import jax
import jax.numpy as jnp
from jax import lax
from jax.experimental import pallas as pl
from jax.experimental.pallas import tpu as pltpu
from jax.experimental.pallas import tpu_sc as plsc

D = 256
C = 4
N = 4096
L = 16
VOL = D * D * D


def _sc_partials(fix1, mov1, fpT, npT):
    info = plsc.get_sparse_core_info()
    nw = info.num_cores * info.num_subcores
    ppw = N // nw
    groups = ppw // L
    mesh = plsc.VectorSubcoreMesh(core_axis_name="c", subcore_axis_name="s")

    @functools.partial(
        pl.kernel,
        out_type=jax.ShapeDtypeStruct((nw, L), jnp.float32),
        mesh=mesh,
        scratch_types=[
            pltpu.VMEM((3 * ppw,), jnp.int32),
            pltpu.VMEM((3 * ppw,), jnp.int32),
            pltpu.VMEM((C, ppw), jnp.int32),
            pltpu.VMEM((C, ppw), jnp.int32),
            pltpu.VMEM((C * ppw,), jnp.float32),
            pltpu.VMEM((C * ppw,), jnp.float32),
            pltpu.VMEM((L,), jnp.float32),
            pltpu.SemaphoreType.DMA,
        ],
    )
    def k(fix_hbm, mov_hbm, fp_hbm, np_hbm, out_hbm,
          fp_v, np_v, idxf_v, idxn_v, dataf_v, datan_v, acc_v, sem):
        wid = lax.axis_index("s") * info.num_cores + lax.axis_index("c")
        base = wid * ppw
        for r in range(3):
            pltpu.sync_copy(fp_hbm.at[pl.ds(r * N + base, ppw)],
                            fp_v.at[pl.ds(r * ppw, ppw)])
            pltpu.sync_copy(np_hbm.at[pl.ds(r * N + base, ppw)],
                            np_v.at[pl.ds(r * ppw, ppw)])

        for g in range(groups):
            for pts, idx_ref in ((fp_v, idxf_v), (np_v, idxn_v)):
                x = pts[pl.ds(0 * ppw + g * L, L)]
                y = pts[pl.ds(1 * ppw + g * L, L)]
                z = pts[pl.ds(2 * ppw + g * L, L)]
                lin = x * (D * D) + y * D + z
                for c in range(C):
                    idx_ref[c, pl.ds(g * L, L)] = lin + c * VOL

        copies = []
        for c in range(C):
            copies.append(pltpu.async_copy(
                fix_hbm.at[idxf_v.at[c]], dataf_v.at[pl.ds(c * ppw, ppw)], sem))
            copies.append(pltpu.async_copy(
                mov_hbm.at[idxn_v.at[c]], datan_v.at[pl.ds(c * ppw, ppw)], sem))
        for cp in copies:
            cp.wait()

        acc = jnp.zeros((L,), jnp.float32)
        for g in range(groups):
            for c in range(C):
                f = dataf_v[pl.ds(c * ppw + g * L, L)]
                m = datan_v[pl.ds(c * ppw + g * L, L)]
                d = f - m
                acc = acc + d * d
        acc_v[...] = acc
        pltpu.sync_copy(acc_v, out_hbm.at[wid])

    return k(fix1, mov1, fpT, npT)


def _finalize_kernel(p_ref, o_ref):
    s = jnp.sum(p_ref[...])
    loss = (0.01 * N - s) * (10000.0 / (2.0 * N))
    o_ref[...] = jnp.broadcast_to(loss, (1, 1))


def kernel(fix_image_feature, moving_image_feature, fixed_points,
           positive_points, negative_points):
    del positive_points
    fix1 = fix_image_feature.reshape(C * VOL)
    mov1 = moving_image_feature.reshape(C * VOL)
    partials = _sc_partials(fix1, mov1, fixed_points.T.reshape(-1),
                            negative_points.T.reshape(-1))
    loss = pl.pallas_call(
        _finalize_kernel,
        out_shape=jax.ShapeDtypeStruct((1, 1), jnp.float32),
    )(partials)
    return loss[0, 0]


# speedup vs baseline: 9.9816x; 9.8518x over previous
import functools

import jax
import jax.numpy as jnp
from jax import lax
from jax.experimental import pallas as pl
from jax.experimental.pallas import tpu as pltpu
from jax.experimental.pallas import tpu_sc as plsc

D = 256
C = 4
N = 4096
L = 16
NROWS = C * D * D

_GATHER_DNUMS = jax.lax.GatherDimensionNumbers(
    offset_dims=(), collapsed_slice_dims=(0,), start_index_map=(0,))


def _lane_pick(vec, lane_vec):
    return lax.gather(vec, lane_vec[:, None], _GATHER_DNUMS, (1,),
                      mode=jax.lax.GatherScatterMode.PROMISE_IN_BOUNDS)


def _sc_partials(fix5, mov5, fpT, npT):
    info = plsc.get_sparse_core_info()
    nw = info.num_cores * info.num_subcores
    ppw = N // nw
    groups = ppw // L
    mesh = plsc.VectorSubcoreMesh(core_axis_name="c", subcore_axis_name="s")

    @functools.partial(
        pl.kernel,
        out_type=jax.ShapeDtypeStruct((nw, L), jnp.float32),
        mesh=mesh,
        scratch_types=[
            pltpu.VMEM((3 * ppw,), jnp.int32),
            pltpu.VMEM((3 * ppw,), jnp.int32),
            pltpu.VMEM((C, ppw), jnp.int32),
            pltpu.VMEM((C, ppw), jnp.int32),
            pltpu.VMEM((ppw, D), jnp.float32),
            pltpu.VMEM((ppw, D), jnp.float32),
            pltpu.VMEM((L,), jnp.float32),
            pltpu.SemaphoreType.DMA,
        ],
    )
    def k(fix5_hbm, mov5_hbm, fp_hbm, np_hbm, out_hbm,
          fp_v, np_v, rowf_v, rown_v, dataf_v, datan_v, acc_v, sem):
        fix_rows = fix5_hbm.reshape(NROWS, D)
        mov_rows = mov5_hbm.reshape(NROWS, D)
        wid = lax.axis_index("s") * info.num_cores + lax.axis_index("c")
        base = wid * ppw
        for r in range(3):
            pltpu.sync_copy(fp_hbm.at[pl.ds(r * N + base, ppw)],
                            fp_v.at[pl.ds(r * ppw, ppw)])
            pltpu.sync_copy(np_hbm.at[pl.ds(r * N + base, ppw)],
                            np_v.at[pl.ds(r * ppw, ppw)])

        for g in range(groups):
            for pts, row_ref in ((fp_v, rowf_v), (np_v, rown_v)):
                x = pts[pl.ds(0 * ppw + g * L, L)]
                y = pts[pl.ds(1 * ppw + g * L, L)]
                row = x * D + y
                for c in range(C):
                    row_ref[c, pl.ds(g * L, L)] = row + c * (D * D)

        lanes = lax.iota(jnp.int32, L)

        def round_body(c, acc):
            cpf = pltpu.async_copy(fix_rows.at[rowf_v.at[c]], dataf_v, sem)
            cpn = pltpu.async_copy(mov_rows.at[rown_v.at[c]], datan_v, sem)
            cpf.wait()
            cpn.wait()
            for g in range(groups):
                zf_vec = fp_v[pl.ds(2 * ppw + g * L, L)]
                zn_vec = np_v[pl.ds(2 * ppw + g * L, L)]
                bf_vec = lax.bitwise_and(zf_vec, L - 1)
                bn_vec = lax.bitwise_and(zn_vec, L - 1)
                for i in range(L):
                    p = g * L + i
                    zf = zf_vec[i]
                    zn = zn_vec[i]
                    chf = dataf_v[p, pl.ds((zf >> 4) * L, L)]
                    chn = datan_v[p, pl.ds((zn >> 4) * L, L)]
                    fsp = _lane_pick(chf, jnp.full((L,), bf_vec[i], jnp.int32))
                    msp = _lane_pick(chn, jnp.full((L,), bn_vec[i], jnp.int32))
                    dd = (fsp - msp) * (fsp - msp)
                    acc = acc + jnp.where(lanes == i, dd, 0.0)
            return acc

        acc = lax.fori_loop(0, C, round_body, jnp.zeros((L,), jnp.float32))
        acc_v[...] = acc
        pltpu.sync_copy(acc_v, out_hbm.at[wid])

    return k(fix5, mov5, fpT, npT)


def _finalize_kernel(p_ref, o_ref):
    s = jnp.sum(p_ref[...])
    loss = (0.01 * N - s) * (10000.0 / (2.0 * N))
    o_ref[...] = jnp.broadcast_to(loss, (1, 1))


def kernel(fix_image_feature, moving_image_feature, fixed_points,
           positive_points, negative_points):
    del positive_points
    partials = _sc_partials(fix_image_feature, moving_image_feature,
                            fixed_points.T.reshape(-1),
                            negative_points.T.reshape(-1))
    loss = pl.pallas_call(
        _finalize_kernel,
        out_shape=jax.ShapeDtypeStruct((1, 1), jnp.float32),
    )(partials)
    return loss[0, 0]
